# initial kernel scaffold (unmeasured)
import jax
import jax.numpy as jnp
from jax import lax
from jax.experimental import pallas as pl
from jax.experimental.pallas import tpu as pltpu


def kernel(
    x,
):
    def body(*refs):
        pass

    out_shape = jax.ShapeDtypeStruct(..., jnp.float32)
    return pl.pallas_call(body, out_shape=out_shape)(...)


# baseline (device time: 30345 ns/iter reference)
import jax
import jax.numpy as jnp
from jax import lax
from jax.experimental import pallas as pl
from jax.experimental.pallas import tpu as pltpu


def kernel(x):
    m, n = x.shape
    half = n // 2

    def body(x_ref, out_ref, send_buf, recv_buf, send_sem, recv_sem):
        my_x = lax.axis_index("x")
        my_y = lax.axis_index("y")
        my_z = lax.axis_index("z")
        other = 1 - my_x

        barrier = pltpu.get_barrier_semaphore()
        pl.semaphore_signal(
            barrier, inc=1,
            device_id=(other, my_y, my_z),
            device_id_type=pl.DeviceIdType.MESH,
        )
        pl.semaphore_wait(barrier, 1)

        @pl.when(my_x == 0)
        def _():
            send_buf[...] = x_ref[:, half:]

        @pl.when(my_x == 1)
        def _():
            send_buf[...] = x_ref[:, :half]

        rdma = pltpu.make_async_remote_copy(
            src_ref=send_buf,
            dst_ref=recv_buf,
            send_sem=send_sem,
            recv_sem=recv_sem,
            device_id=(other, my_y, my_z),
            device_id_type=pl.DeviceIdType.MESH,
        )
        rdma.start()

        @pl.when(my_x == 0)
        def _():
            out_ref[:m, :] = x_ref[:, :half]

        @pl.when(my_x == 1)
        def _():
            out_ref[m:, :] = x_ref[:, half:]

        rdma.wait()

        @pl.when(my_x == 0)
        def _():
            out_ref[m:, :] = recv_buf[...]

        @pl.when(my_x == 1)
        def _():
            out_ref[:m, :] = recv_buf[...]

    return pl.pallas_call(
        body,
        out_shape=jax.ShapeDtypeStruct((2 * m, half), x.dtype),
        in_specs=[pl.BlockSpec(memory_space=pltpu.VMEM)],
        out_specs=pl.BlockSpec(memory_space=pltpu.VMEM),
        scratch_shapes=[
            pltpu.VMEM((m, half), x.dtype),
            pltpu.VMEM((m, half), x.dtype),
            pltpu.SemaphoreType.DMA,
            pltpu.SemaphoreType.DMA,
        ],
        compiler_params=pltpu.CompilerParams(collective_id=0),
    )(x)


# device time: 23659 ns/iter; 1.2826x vs baseline; 1.2826x over previous
import jax
import jax.numpy as jnp
from jax import lax
from jax.experimental import pallas as pl
from jax.experimental.pallas import tpu as pltpu

C = 8


def kernel(x):
    m, n = x.shape
    half = n // 2
    rows_half = m // 2
    ch = rows_half // C

    def body(x_ref, out_ref, send_buf, xrecv_buf,
             stage_sems, x_send_sems, x_recv_sems, y_send_sems, y_recv_sems,
             fwd_sems, loc_sem):
        my_x = lax.axis_index("x")
        my_y = lax.axis_index("y")
        my_z = lax.axis_index("z")
        ox = 1 - my_x
        oy = 1 - my_y
        xp = (ox, my_y, my_z)
        yp = (my_x, oy, my_z)

        barrier = pltpu.get_barrier_semaphore()
        for nbr in (xp, yp):
            pl.semaphore_signal(
                barrier, inc=1,
                device_id=nbr, device_id_type=pl.DeviceIdType.MESH,
            )
        pl.semaphore_wait(barrier, 2)

        send_base = my_y * rows_half
        x_dst_base = my_x * m + my_y * rows_half
        fwd_base = ox * m + my_y * rows_half

        loc = pltpu.make_async_copy(
            x_ref.at[:, pl.ds(my_x * half, half)],
            out_ref.at[pl.ds(my_x * m, m), :],
            loc_sem,
        )
        loc.start()

        stages = []
        x_rdmas = []
        y_rdmas = []
        fwds = []
        for c in range(C):
            rows = pl.ds(send_base + c * ch, ch)
            stages.append(pltpu.make_async_copy(
                x_ref.at[rows, pl.ds(ox * half, half)],
                send_buf.at[c * ch:(c + 1) * ch, :],
                stage_sems.at[c],
            ))
            x_rdmas.append(pltpu.make_async_remote_copy(
                src_ref=send_buf.at[c * ch:(c + 1) * ch, :],
                dst_ref=xrecv_buf.at[c * ch:(c + 1) * ch, :],
                send_sem=x_send_sems.at[c],
                recv_sem=x_recv_sems.at[c],
                device_id=xp,
                device_id_type=pl.DeviceIdType.MESH,
            ))
            y_rdmas.append(pltpu.make_async_remote_copy(
                src_ref=xrecv_buf.at[c * ch:(c + 1) * ch, :],
                dst_ref=out_ref.at[pl.ds(fwd_base + c * ch, ch), :],
                send_sem=y_send_sems.at[c],
                recv_sem=y_recv_sems.at[c],
                device_id=yp,
                device_id_type=pl.DeviceIdType.MESH,
            ))
            fwds.append(pltpu.make_async_copy(
                xrecv_buf.at[c * ch:(c + 1) * ch, :],
                out_ref.at[pl.ds(fwd_base + c * ch, ch), :],
                fwd_sems.at[c],
            ))

        for s in stages:
            s.start()
        for c in range(C):
            stages[c].wait()
            x_rdmas[c].start()
        for c in range(C):
            x_rdmas[c].wait_recv()
            y_rdmas[c].start()
            fwds[c].start()

        loc.wait()
        for c in range(C):
            x_rdmas[c].wait_send()
            fwds[c].wait()
            y_rdmas[c].wait()

    return pl.pallas_call(
        body,
        out_shape=jax.ShapeDtypeStruct((2 * m, half), x.dtype),
        in_specs=[pl.BlockSpec(memory_space=pl.ANY)],
        out_specs=pl.BlockSpec(memory_space=pl.ANY),
        scratch_shapes=[
            pltpu.VMEM((rows_half, half), x.dtype),
            pltpu.VMEM((rows_half, half), x.dtype),
            pltpu.SemaphoreType.DMA((C,)),
            pltpu.SemaphoreType.DMA((C,)),
            pltpu.SemaphoreType.DMA((C,)),
            pltpu.SemaphoreType.DMA((C,)),
            pltpu.SemaphoreType.DMA((C,)),
            pltpu.SemaphoreType.DMA((C,)),
            pltpu.SemaphoreType.DMA,
        ],
        compiler_params=pltpu.CompilerParams(collective_id=0),
    )(x)
